# Initial kernel scaffold; baseline (speedup 1.0000x reference)
import jax
import jax.numpy as jnp
from jax.experimental import pallas as pl


def kernel(x, edge_index, batch, atom_emb, Wb, Wc, bc, cb, gam, bet, W1, g1, be1, W2, g2, be2, W3, b3):
    raise NotImplementedError("write your pallas kernel here")


# speedup vs baseline: 9.9003x; 9.9003x over previous
import functools

import jax
import jax.numpy as jnp
from jax import lax
from jax.experimental import pallas as pl
from jax.experimental.pallas import tpu as pltpu
from jax.experimental.pallas import tpu_sc as plsc

N = 10000
E = 320000
H = 128
HEADS = 8
BASES = 4
LAYERS = 4
G = 64
D = H // HEADS
W = BASES * D

NC = 2
NS = 16
NW = NC * NS
CH = 128
NPAD = 10112
RPT = NPAD // NS
EPAD = NW * 79 * CH
APAD = NW * 22 * CH
AZ = 9 * 119

BR = 1264
NB = NPAD // BR

f32 = jnp.float32
i32 = jnp.int32


def _sc_scatter_body(tab, gidx, sidx, outp,
                     gi0, si0, rows0, gi1, si1, rows1, acc, sem0, sem1):
    c = lax.axis_index("c")
    s = lax.axis_index("s")
    wid = s * NC + c
    r0 = s * RPT
    nk = gidx.shape[0] // (NW * CH)
    ebase = wid * (nk * CH)

    @pl.loop(0, CH)
    def _(i):
        z = jnp.zeros((16,), f32)
        for j in range(W // 16):
            rows0[i, pl.ds(j * 16, 16)] = z

    for t in range(RPT // CH + 1):
        sz = min(CH, RPT - t * CH)
        pltpu.sync_copy(rows0.at[pl.ds(0, sz)], acc.at[pl.ds(r0 + t * CH, sz)])
    plsc.subcore_barrier()

    bufs = ((gi0, si0, rows0, sem0), (gi1, si1, rows1, sem1))

    pltpu.sync_copy(gidx.at[pl.ds(ebase, CH)], gi0)
    pltpu.sync_copy(sidx.at[pl.ds(ebase, CH)], si0)
    pltpu.async_copy(tab.at[gi0], rows0, sem0)

    @pl.loop(0, nk, step=2)
    def _(k):
        for p in range(2):
            gip, sip, rowsp, semp = bufs[p]
            giq, siq, rowsq, semq = bufs[1 - p]
            kk = k + p

            @pl.when(kk < nk)
            def _():
                pltpu.make_async_copy(tab.at[gip], rowsp, semp).wait()

                @pl.when(kk + 1 < nk)
                def _():
                    nbase = ebase + (kk + 1) * CH
                    pltpu.sync_copy(gidx.at[pl.ds(nbase, CH)], giq)
                    pltpu.sync_copy(sidx.at[pl.ds(nbase, CH)], siq)
                    pltpu.async_copy(tab.at[giq], rowsq, semq)

                pltpu.sync_copy(rowsp, acc.at[sip], add=True)

    plsc.subcore_barrier()
    for t in range(RPT // CH + 1):
        sz = min(CH, RPT - t * CH)
        pltpu.sync_copy(acc.at[pl.ds(r0 + t * CH, sz)], rows0.at[pl.ds(0, sz)])
        pltpu.sync_copy(rows0.at[pl.ds(0, sz)], outp.at[c, pl.ds(r0 + t * CH, sz)])


@functools.lru_cache(None)
def _sc_scatter():
    mesh = plsc.VectorSubcoreMesh(core_axis_name="c", subcore_axis_name="s",
                                  num_cores=NC, num_subcores=NS)
    return pl.kernel(
        _sc_scatter_body,
        out_type=jax.ShapeDtypeStruct((NC, NPAD, W), f32),
        mesh=mesh,
        compiler_params=pltpu.CompilerParams(use_tc_tiling_on_sc=False),
        scratch_types=[
            pltpu.VMEM((CH,), i32),
            pltpu.VMEM((CH,), i32),
            pltpu.VMEM((CH, W), f32),
            pltpu.VMEM((CH,), i32),
            pltpu.VMEM((CH,), i32),
            pltpu.VMEM((CH, W), f32),
            pltpu.VMEM_SHARED((NPAD, W), f32),
            pltpu.SemaphoreType.DMA,
            pltpu.SemaphoreType.DMA,
        ],
    )


def _blk_mask(i):
    return (lax.broadcasted_iota(i32, (BR, 1), 0) + i * BR < N).astype(f32)


def _tc_pre_body(h0pl, h0ph, degp, wb, wc, bcn, h_o, dinv_o, btab_o, comb_o):
    i = pl.program_id(0)
    h = jnp.concatenate([h0pl[0] + h0pl[1], h0ph[0] + h0ph[1]], axis=1)
    deg = degp[0][:, 0:1] + degp[1][:, 0:1] + 1.0
    dinv = lax.rsqrt(deg)
    h_o[...] = h
    dinv_o[...] = dinv
    btab_o[...] = jnp.dot(h, wb[...], preferred_element_type=f32) * dinv * _blk_mask(i)
    comb_o[...] = jnp.dot(h, wc[...], preferred_element_type=f32) + bcn[...]


def _spread_mats():
    r32 = lax.broadcasted_iota(i32, (HEADS * BASES, H), 0)
    c32 = lax.broadcasted_iota(i32, (HEADS * BASES, H), 1)
    r64 = lax.broadcasted_iota(i32, (W, H), 0)
    c64 = lax.broadcasted_iota(i32, (W, H), 1)
    S = [(r32 == (c32 // D) * BASES + b).astype(f32) for b in range(BASES)]
    T = [(r64 == b * D + c64 % D).astype(f32) for b in range(BASES)]
    return S, T


def _tc_stats_body(aggp, btab, comb, dinv, cbl, out_o, stats_o, acc):
    i = pl.program_id(0)

    @pl.when(i == 0)
    def _():
        acc[...] = jnp.zeros_like(acc)

    agg = (aggp[0] + aggp[1] + btab[...]) * dinv[...]
    cmb = comb[...]
    S, T = _spread_mats()
    out = jnp.dot(cmb, S[0], preferred_element_type=f32) * \
        jnp.dot(agg, T[0], preferred_element_type=f32)
    for b in range(1, BASES):
        out = out + jnp.dot(cmb, S[b], preferred_element_type=f32) * \
            jnp.dot(agg, T[b], preferred_element_type=f32)
    out = out + cbl[...]
    out_o[...] = out
    om = out * _blk_mask(i)
    acc[0:1] = acc[0:1] + jnp.sum(om, axis=0, keepdims=True)
    acc[1:2] = acc[1:2] + jnp.sum(om * om, axis=0, keepdims=True)
    stats_o[...] = acc[...]


def _tc_update_body(last, *refs):
    if last:
        (out_r, stats, h_r, gaml, betl, hn_o) = refs
    else:
        (out_r, stats, h_r, dinv_r, wb, wc, bcn, gaml, betl,
         hn_o, btab_o, comb_o) = refs
    i = pl.program_id(0)
    mu = stats[0:1] * (1.0 / N)
    var = stats[1:2] * (1.0 / N) - mu * mu
    xn = (out_r[...] - mu) * lax.rsqrt(var + 1e-5) * gaml[...] + betl[...]
    hn = h_r[...] + jnp.maximum(xn, 0.0)
    hn_o[...] = hn
    if not last:
        dinv = dinv_r[...]
        btab_o[...] = jnp.dot(hn, wb[...], preferred_element_type=f32) * dinv * _blk_mask(i)
        comb_o[...] = jnp.dot(hn, wc[...], preferred_element_type=f32) + bcn[...]


def _tc_head_body(h_r, batchp, w1, g1, be1, w2, g2, be2, w3, b3, z_o):
    h = h_r[...]
    onehot = (batchp[...] == lax.broadcasted_iota(i32, (NPAD, G), 1)).astype(f32)
    dn = (((0,), (0,)), ((), ()))
    cnt = lax.dot_general(onehot, jnp.ones((NPAD, 1), f32), dn,
                          preferred_element_type=f32)
    pooled = lax.dot_general(onehot, h, dn, preferred_element_type=f32)
    pooled = pooled / jnp.maximum(cnt, 1.0)

    def bn_relu(z, g, b):
        mu = jnp.mean(z, axis=0, keepdims=True)
        zc = z - mu
        var = jnp.mean(zc * zc, axis=0, keepdims=True)
        return jnp.maximum(zc * lax.rsqrt(var + 1e-5) * g[...] + b[...], 0.0)

    z = bn_relu(jnp.dot(pooled, w1[...], preferred_element_type=f32), g1, be1)
    z = bn_relu(jnp.dot(z, w2[...], preferred_element_type=f32), g2, be2)
    z_o[...] = jnp.dot(z, w3[...], preferred_element_type=f32) + b3[...]


def _rows(shape):
    return pl.BlockSpec(shape, lambda i: (0,) * len(shape))


def _rblk(width):
    return pl.BlockSpec((BR, width), lambda i: (i, 0))


def _rblk3(width):
    return pl.BlockSpec((2, BR, width), lambda i: (0, i, 0))


@functools.lru_cache(None)
def _tc_pre():
    return pl.pallas_call(
        _tc_pre_body,
        grid=(NB,),
        in_specs=[_rblk3(W), _rblk3(W), _rblk3(W),
                  _rows((H, W)), _rows((H, HEADS * BASES)), _rows((1, HEADS * BASES))],
        out_specs=[_rblk(H), _rblk(1), _rblk(W), _rblk(HEADS * BASES)],
        out_shape=[jax.ShapeDtypeStruct((NPAD, H), f32),
                   jax.ShapeDtypeStruct((NPAD, 1), f32),
                   jax.ShapeDtypeStruct((NPAD, W), f32),
                   jax.ShapeDtypeStruct((NPAD, HEADS * BASES), f32)],
    )


@functools.lru_cache(None)
def _tc_stats():
    return pl.pallas_call(
        _tc_stats_body,
        grid=(NB,),
        in_specs=[_rblk3(W), _rblk(W), _rblk(HEADS * BASES), _rblk(1),
                  _rows((1, H))],
        out_specs=[_rblk(H), _rows((2, H))],
        out_shape=[jax.ShapeDtypeStruct((NPAD, H), f32),
                   jax.ShapeDtypeStruct((2, H), f32)],
        scratch_shapes=[pltpu.VMEM((2, H), f32)],
    )


@functools.lru_cache(None)
def _tc_update(last):
    if last:
        in_specs = [_rblk(H), _rows((2, H)), _rblk(H),
                    _rows((1, H)), _rows((1, H))]
        out_specs = _rblk(H)
        outs = jax.ShapeDtypeStruct((NPAD, H), f32)
    else:
        in_specs = [_rblk(H), _rows((2, H)), _rblk(H), _rblk(1),
                    _rows((H, W)), _rows((H, HEADS * BASES)), _rows((1, HEADS * BASES)),
                    _rows((1, H)), _rows((1, H))]
        out_specs = [_rblk(H), _rblk(W), _rblk(HEADS * BASES)]
        outs = [jax.ShapeDtypeStruct((NPAD, H), f32),
                jax.ShapeDtypeStruct((NPAD, W), f32),
                jax.ShapeDtypeStruct((NPAD, HEADS * BASES), f32)]
    return pl.pallas_call(functools.partial(_tc_update_body, last),
                          grid=(NB,), in_specs=in_specs, out_specs=out_specs,
                          out_shape=outs)


@functools.lru_cache(None)
def _tc_head():
    return pl.pallas_call(
        _tc_head_body,
        out_shape=jax.ShapeDtypeStruct((G, 1), f32),
    )


def _impl(x, edge_index, batch, atom_emb, Wb, Wc, bc, cb, gam, bet,
          W1, g1, be1, W2, g2, be2, W3, b3):
    atab = jnp.concatenate([atom_emb.reshape(AZ, H),
                            jnp.zeros((1, H), f32)], axis=0)
    aidx = (x.astype(i32) + (jnp.arange(9, dtype=i32) * 119)[None, :]).reshape(-1)
    aidx = jnp.concatenate([aidx, jnp.full((APAD - 9 * N,), AZ, i32)])
    anode = jnp.repeat(jnp.arange(N, dtype=i32), 9)
    anode = jnp.concatenate([anode, jnp.full((APAD - 9 * N,), N, i32)])
    epad = jnp.full((EPAD - E,), N, i32)
    srcp = jnp.concatenate([edge_index[0], epad])
    dstp = jnp.concatenate([edge_index[1], epad])
    batchp = jnp.concatenate([batch.astype(i32),
                              jnp.full((NPAD - N,), G, i32)]).reshape(NPAD, 1)

    sc = _sc_scatter()
    h0pl = sc(atab[:, :W], aidx, anode)
    h0ph = sc(atab[:, W:], aidx, anode)
    degp = sc(jnp.ones((NPAD, W), f32), dstp, dstp)
    h, dinv, btab, comb = _tc_pre()(h0pl, h0ph, degp, Wb[0], Wc[0], bc[0][None])
    for l in range(LAYERS):
        aggp = sc(btab, srcp, dstp)
        out, stats = _tc_stats()(aggp, btab, comb, dinv, cb[l][None])
        last = l == LAYERS - 1
        if last:
            h = _tc_update(True)(out, stats, h, gam[l][None], bet[l][None])
        else:
            h, btab, comb = _tc_update(False)(
                out, stats, h, dinv, Wb[l + 1], Wc[l + 1], bc[l + 1][None],
                gam[l][None], bet[l][None])
    return _tc_head()(h, batchp, W1, g1[None], be1[None],
                      W2, g2[None], be2[None], W3, b3[None])


_kernel_jit = jax.jit(_impl)


def kernel(x, edge_index, batch, atom_emb, Wb, Wc, bc, cb, gam, bet,
           W1, g1, be1, W2, g2, be2, W3, b3):
    return _kernel_jit(x, edge_index, batch, atom_emb, Wb, Wc, bc, cb, gam, bet,
                       W1, g1, be1, W2, g2, be2, W3, b3)
